# Initial kernel scaffold; baseline (speedup 1.0000x reference)
import jax
import jax.numpy as jnp
from jax.experimental import pallas as pl


def kernel(x, edge_index, W1, b1, W2, b2):
    raise NotImplementedError("write your pallas kernel here")


# speedup vs baseline: 210.4448x; 210.4448x over previous
import functools

import jax
import jax.numpy as jnp
from jax import lax
from jax.experimental import pallas as pl
from jax.experimental.pallas import tpu as pltpu
from jax.experimental.pallas import tpu_sc as plsc

N = 100000
E = 6400000
NC = 2
NS = 16
NW = NC * NS
NP = 102400
SUB = NP // NS
EW = E // NW
CH = 8000
ITERS = EW // CH


def _mesh():
    return plsc.VectorSubcoreMesh(core_axis_name="c", subcore_axis_name="s")


def _hist(dst, zeros1, ones1):
    @functools.partial(
        pl.kernel,
        out_type=jax.ShapeDtypeStruct((NC, NP), jnp.float32),
        mesh=_mesh(),
        compiler_params=pltpu.CompilerParams(use_tc_tiling_on_sc=False),
        scratch_types=[
            pltpu.VMEM((CH,), jnp.int32),
            pltpu.VMEM((CH,), jnp.float32),
            pltpu.VMEM((SUB,), jnp.float32),
            pltpu.VMEM_SHARED((NP,), jnp.float32),
        ],
    )
    def k(dst_hbm, z_hbm, ones_hbm, out_hbm, idx_v, ones_v, buf_v, acc_sh):
        cid = lax.axis_index("c")
        sid = lax.axis_index("s")
        wid = cid * NS + sid
        pltpu.sync_copy(ones_hbm, ones_v)
        pltpu.sync_copy(z_hbm.at[pl.ds(sid * SUB, SUB)], buf_v)
        pltpu.sync_copy(buf_v, acc_sh.at[pl.ds(sid * SUB, SUB)])
        plsc.subcore_barrier()

        def body(i, _):
            base = wid * EW + i * CH
            pltpu.sync_copy(dst_hbm.at[pl.ds(base, CH)], idx_v)
            pltpu.sync_copy(ones_v, acc_sh.at[idx_v], add=True)
            return 0

        lax.fori_loop(0, ITERS, body, 0)
        plsc.subcore_barrier()
        pltpu.sync_copy(acc_sh.at[pl.ds(sid * SUB, SUB)], buf_v)
        pltpu.sync_copy(buf_v, out_hbm.at[cid, pl.ds(sid * SUB, SUB)])

    return k(dst, zeros1, ones1)


def _msgpass(src, dst, t0, t1, zeros1):
    @functools.partial(
        pl.kernel,
        out_type=jax.ShapeDtypeStruct((NC, 2, NP), jnp.float32),
        mesh=_mesh(),
        compiler_params=pltpu.CompilerParams(use_tc_tiling_on_sc=False),
        scratch_types=[
            pltpu.VMEM((CH,), jnp.int32),
            pltpu.VMEM((CH,), jnp.int32),
            pltpu.VMEM((CH,), jnp.float32),
            pltpu.VMEM((CH,), jnp.float32),
            pltpu.VMEM((SUB,), jnp.float32),
            pltpu.VMEM_SHARED((NP,), jnp.float32),
            pltpu.VMEM_SHARED((NP,), jnp.float32),
            pltpu.VMEM_SHARED((NP,), jnp.float32),
            pltpu.VMEM_SHARED((NP,), jnp.float32),
        ],
    )
    def k(src_hbm, dst_hbm, t0_hbm, t1_hbm, z_hbm, out_hbm, idxs_v, idxd_v,
          m0_v, m1_v, buf_v, t0_sh, t1_sh, a0_sh, a1_sh):
        cid = lax.axis_index("c")
        sid = lax.axis_index("s")
        wid = cid * NS + sid
        sl = pl.ds(sid * SUB, SUB)
        pltpu.sync_copy(z_hbm.at[sl], buf_v)
        pltpu.sync_copy(buf_v, a0_sh.at[sl])
        pltpu.sync_copy(buf_v, a1_sh.at[sl])
        pltpu.sync_copy(t0_hbm.at[sl], buf_v)
        pltpu.sync_copy(buf_v, t0_sh.at[sl])
        pltpu.sync_copy(t1_hbm.at[sl], buf_v)
        pltpu.sync_copy(buf_v, t1_sh.at[sl])
        plsc.subcore_barrier()

        def body(i, _):
            base = wid * EW + i * CH
            pltpu.sync_copy(src_hbm.at[pl.ds(base, CH)], idxs_v)
            pltpu.sync_copy(dst_hbm.at[pl.ds(base, CH)], idxd_v)
            pltpu.sync_copy(t0_sh.at[idxs_v], m0_v)
            pltpu.sync_copy(t1_sh.at[idxs_v], m1_v)
            pltpu.sync_copy(m0_v, a0_sh.at[idxd_v], add=True)
            pltpu.sync_copy(m1_v, a1_sh.at[idxd_v], add=True)
            return 0

        lax.fori_loop(0, ITERS, body, 0)
        plsc.subcore_barrier()
        pltpu.sync_copy(a0_sh.at[sl], buf_v)
        pltpu.sync_copy(buf_v, out_hbm.at[cid, 0, sl])
        pltpu.sync_copy(a1_sh.at[sl], buf_v)
        pltpu.sync_copy(buf_v, out_hbm.at[cid, 1, sl])

    return k(src, dst, t0, t1, zeros1)


def _tc_pre(degp, xT, W1):
    def body(degp_ref, xT_ref, w_ref, dinv_ref, gT_ref):
        deg = degp_ref[0:1, :] + degp_ref[1:2, :] + 1.0
        dinv = lax.rsqrt(deg)
        dinv_ref[...] = dinv
        x0 = xT_ref[0:1, :]
        x1 = xT_ref[1:2, :]
        g0 = dinv * (x0 * w_ref[0:1, 0:1] + x1 * w_ref[1:2, 0:1])
        g1 = dinv * (x0 * w_ref[0:1, 1:2] + x1 * w_ref[1:2, 1:2])
        gT_ref[...] = jnp.concatenate([g0, g1], axis=0)

    return pl.pallas_call(
        body,
        out_shape=[
            jax.ShapeDtypeStruct((1, NP), jnp.float32),
            jax.ShapeDtypeStruct((2, NP), jnp.float32),
        ],
    )(degp, xT, W1)


def _tc_mid(Sa, Sb, gT, dinv, W2, b1):
    def body(sa_ref, sb_ref, gT_ref, dinv_ref, w_ref, b_ref, g2T_ref):
        dinv = dinv_ref[...]
        o = dinv * (sa_ref[...] + sb_ref[...] + gT_ref[...]) + b_ref[...]
        o0 = o[0:1, :]
        o1 = o[1:2, :]
        g0 = dinv * (o0 * w_ref[0:1, 0:1] + o1 * w_ref[1:2, 0:1])
        g1 = dinv * (o0 * w_ref[0:1, 1:2] + o1 * w_ref[1:2, 1:2])
        g2T_ref[...] = jnp.concatenate([g0, g1], axis=0)

    return pl.pallas_call(
        body,
        out_shape=jax.ShapeDtypeStruct((2, NP), jnp.float32),
    )(Sa, Sb, gT, dinv, W2, b1)


def _tc_post(Sa, Sb, gT, dinv, b2):
    def body(sa_ref, sb_ref, gT_ref, dinv_ref, b_ref, out_ref):
        out_ref[...] = (
            dinv_ref[...] * (sa_ref[...] + sb_ref[...] + gT_ref[...])
            + b_ref[...]
        )

    return pl.pallas_call(
        body,
        out_shape=jax.ShapeDtypeStruct((2, NP), jnp.float32),
    )(Sa, Sb, gT, dinv, b2)


def kernel(x, edge_index, W1, b1, W2, b2):
    assert x.shape == (N, 2) and edge_index.shape == (2, E)
    src = edge_index[0]
    dst = edge_index[1]
    xT = jnp.pad(x, ((0, NP - N), (0, 0))).T
    b1c = jnp.broadcast_to(b1.reshape(2, 1), (2, NP))
    b2c = jnp.broadcast_to(b2.reshape(2, 1), (2, NP))
    zeros1 = jnp.zeros((NP,), jnp.float32)
    ones1 = jnp.ones((CH,), jnp.float32)

    degp = _hist(dst, zeros1, ones1)
    dinv, g1T = _tc_pre(degp, xT, W1)

    S1p = _msgpass(src, dst, g1T[0], g1T[1], zeros1)
    g2T = _tc_mid(S1p[0], S1p[1], g1T, dinv, W2, b1c)

    S2p = _msgpass(src, dst, g2T[0], g2T[1], zeros1)
    out2T = _tc_post(S2p[0], S2p[1], g2T, dinv, b2c)

    return out2T.T[:N]
